# baseline (speedup 1.0000x reference)
import functools

import jax
import jax.numpy as jnp
from jax import lax
from jax.experimental import pallas as pl
from jax.experimental.pallas import tpu as pltpu
from jax.experimental.pallas import tpu_sc as plsc

CHAR_VOCAB = 256
CHAR_EMB = 16
WORD_LEN = 20
HIDDEN = 128
OUT_DIM = 64
NPART = 3
PART_CHARS = 8
PAIRS_PER_PART = PART_CHARS // 2
PAIR_DIM = 2 * CHAR_EMB

TOKEN_BLOCK = 2048
SC_CHUNK = 1280
LANES = 16


def _pair_indices(g, grp, chv_ref):
    ql = jax.lax.iota(jnp.int32, LANES) + grp * LANES
    lt = ql >> 2
    j = ql & 3
    if g == 0:
        col = 2 * j
    elif g == 1:
        col = PART_CHARS + 2 * j
    else:
        col = jnp.where(j < 2, 2 * PART_CHARS + 2 * j, 2 * j - 4)
    addr = lt * WORD_LEN + col
    hi = plsc.load_gather(chv_ref, [addr])
    lo = plsc.load_gather(chv_ref, [addr + 1])
    return (hi << 8) | lo


def _make_sc_gather(n_tok):
    info = plsc.get_sparse_core_info()
    nw = info.num_cores * info.num_subcores
    part_pairs = n_tok * PAIRS_PER_PART
    per_w = part_pairs // nw
    steps = per_w // SC_CHUNK
    assert part_pairs % nw == 0 and per_w % SC_CHUNK == 0 and steps % 2 == 0
    tok_chunk = SC_CHUNK // PAIRS_PER_PART
    ch_chunk = tok_chunk * WORD_LEN
    ngrp = SC_CHUNK // LANES
    n_idx = NPART * part_pairs
    mesh = plsc.VectorSubcoreMesh(core_axis_name="c", subcore_axis_name="s")

    @functools.partial(
        pl.kernel,
        mesh=mesh,
        compiler_params=pltpu.CompilerParams(use_tc_tiling_on_sc=False,
                                             needs_layout_passes=False),
        out_type=jax.ShapeDtypeStruct((n_idx, PAIR_DIM), jnp.float32),
        scratch_types=[
            pltpu.VMEM((ch_chunk,), jnp.int32),
            pltpu.VMEM((ch_chunk,), jnp.int32),
            pltpu.VMEM((SC_CHUNK,), jnp.int32),
            pltpu.VMEM((SC_CHUNK,), jnp.int32),
            pltpu.VMEM((SC_CHUNK, PAIR_DIM), jnp.float32),
            pltpu.VMEM((SC_CHUNK, PAIR_DIM), jnp.float32),
            pltpu.SemaphoreType.DMA,
            pltpu.SemaphoreType.DMA,
            pltpu.SemaphoreType.DMA,
            pltpu.SemaphoreType.DMA,
            pltpu.SemaphoreType.DMA,
        ],
    )
    def sc_gather(ch_hbm, table_hbm, out_hbm,
                  chv0, chv1, idxp0, idxp1, rows_v0, rows_v1,
                  sc0, sc1, so0, so1, sg):
        wid = lax.axis_index("s") * info.num_cores + lax.axis_index("c")
        w_ch = wid * steps * ch_chunk

        def ch_slice(c):
            return ch_hbm.at[pl.ds(w_ch + c * ch_chunk, ch_chunk)]

        def out_slice(base):
            return out_hbm.at[pl.ds(base, SC_CHUNK)]

        def compute_idx(g, c, chv, idxp):
            def grp_body(grp, carry):
                idxp[pl.ds(grp * LANES, LANES)] = _pair_indices(g, grp, chv)
                return carry
            lax.fori_loop(0, ngrp, grp_body, 0)

        pltpu.async_copy(rows_v0, out_slice(wid * per_w), so0)
        pltpu.async_copy(rows_v1, out_slice(wid * per_w + SC_CHUNK), so1)

        for g in range(NPART):
            w_out = g * part_pairs + wid * per_w

            pltpu.async_copy(ch_slice(0), chv0, sc0)
            pltpu.async_copy(ch_slice(1), chv1, sc1)

            def body(k, carry, g=g, w_out=w_out):
                c0 = 2 * k
                c1 = c0 + 1
                base0 = w_out + c0 * SC_CHUNK
                base1 = w_out + c1 * SC_CHUNK
                pre0 = jnp.minimum(c0 + 2, steps - 1)
                pre1 = jnp.minimum(c1 + 2, steps - 1)

                pltpu.make_async_copy(ch_slice(c0), chv0, sc0).wait()
                compute_idx(g, c0, chv0, idxp0)
                pltpu.async_copy(ch_slice(pre0), chv0, sc0)
                pltpu.make_async_copy(rows_v0, out_slice(base0), so0).wait()
                pltpu.async_copy(table_hbm.at[idxp0], rows_v0, sg).wait()
                pltpu.async_copy(rows_v0, out_slice(base0), so0)

                pltpu.make_async_copy(ch_slice(c1), chv1, sc1).wait()
                compute_idx(g, c1, chv1, idxp1)
                pltpu.async_copy(ch_slice(pre1), chv1, sc1)
                pltpu.make_async_copy(rows_v1, out_slice(base1), so1).wait()
                pltpu.async_copy(table_hbm.at[idxp1], rows_v1, sg).wait()
                pltpu.async_copy(rows_v1, out_slice(base1), so1)
                return carry

            lax.fori_loop(0, steps // 2, body, 0)

            pltpu.make_async_copy(ch_slice(0), chv0, sc0).wait()
            pltpu.make_async_copy(ch_slice(0), chv1, sc1).wait()

        pltpu.make_async_copy(rows_v0, out_slice(0), so0).wait()
        pltpu.make_async_copy(rows_v1, out_slice(0), so1).wait()

    return sc_gather


def _mlp_kernel(x_ref, w1_ref, b1_ref, w2_ref, b2_ref, out_ref):
    acc = jnp.dot(x_ref[0].astype(jnp.bfloat16), w1_ref[0],
                  preferred_element_type=jnp.float32)
    acc += jnp.dot(x_ref[1].astype(jnp.bfloat16), w1_ref[1],
                   preferred_element_type=jnp.float32)
    acc += jnp.dot(x_ref[2].astype(jnp.bfloat16), w1_ref[2],
                   preferred_element_type=jnp.float32)
    h = jax.nn.relu(acc + b1_ref[...])
    out = jnp.dot(h, w2_ref[...], preferred_element_type=jnp.float32)
    out_ref[...] = out + b2_ref[...]


def kernel(chars, emb, W1, b1, W2, b2):
    b, s, w = chars.shape
    n = b * s

    t2 = jnp.concatenate(
        [jnp.repeat(emb, CHAR_VOCAB, axis=0),
         jnp.tile(emb, (CHAR_VOCAB, 1))], axis=1)

    w1r = jnp.stack(
        [W1[0:128, :], W1[128:256, :],
         jnp.pad(W1[256:, :], ((0, 128 - (w * CHAR_EMB - 256)), (0, 0)))],
        axis=0).astype(jnp.bfloat16)
    b1r = b1.reshape(1, HIDDEN)
    b2r = b2.reshape(1, OUT_DIM)

    nh = n // 2
    sc_gather = _make_sc_gather(nh)
    chars_h = chars.reshape(2, nh * w)
    outs = []
    for r in range(2):
        ce = sc_gather(chars_h[r], t2)
        x3 = ce.reshape(NPART, nh, PART_CHARS * CHAR_EMB)
        grid = (nh // TOKEN_BLOCK,)
        outs.append(pl.pallas_call(
            _mlp_kernel,
            grid=grid,
            in_specs=[
                pl.BlockSpec((NPART, TOKEN_BLOCK, PART_CHARS * CHAR_EMB),
                             lambda i: (0, i, 0)),
                pl.BlockSpec((NPART, 128, HIDDEN), lambda i: (0, 0, 0)),
                pl.BlockSpec((1, HIDDEN), lambda i: (0, 0)),
                pl.BlockSpec((HIDDEN, OUT_DIM), lambda i: (0, 0)),
                pl.BlockSpec((1, OUT_DIM), lambda i: (0, 0)),
            ],
            out_specs=pl.BlockSpec((TOKEN_BLOCK, OUT_DIM), lambda i: (i, 0)),
            out_shape=jax.ShapeDtypeStruct((nh, OUT_DIM), jnp.float32),
        )(x3, w1r, b1r, W2, b2r))

    return jnp.concatenate(outs).reshape(b, s, OUT_DIM)


# speedup vs baseline: 1.0483x; 1.0483x over previous
import functools

import jax
import jax.numpy as jnp
from jax import lax
from jax.experimental import pallas as pl
from jax.experimental.pallas import tpu as pltpu
from jax.experimental.pallas import tpu_sc as plsc

CHAR_VOCAB = 256
CHAR_EMB = 16
WORD_LEN = 20
HIDDEN = 128
OUT_DIM = 64
NPART = 3
PART_CHARS = 8
PAIRS_PER_PART = PART_CHARS // 2
PAIR_DIM = 2 * CHAR_EMB

TOKEN_BLOCK = 2048
SC_CHUNK = 1280
LANES = 16


def _pair_indices(g, grp, chv_ref):
    ql = jax.lax.iota(jnp.int32, LANES) + grp * LANES
    lt = ql >> 2
    j = ql & 3
    if g == 0:
        col = 2 * j
    elif g == 1:
        col = PART_CHARS + 2 * j
    else:
        col = jnp.where(j < 2, 2 * PART_CHARS + 2 * j, 2 * j - 4)
    addr = lt * WORD_LEN + col
    hi = plsc.load_gather(chv_ref, [addr])
    lo = plsc.load_gather(chv_ref, [addr + 1])
    return (hi << 8) | lo


def _make_sc_gather(n_tok):
    info = plsc.get_sparse_core_info()
    nw = info.num_cores * info.num_subcores
    part_pairs = n_tok * PAIRS_PER_PART
    per_w = part_pairs // nw
    steps = per_w // SC_CHUNK
    assert part_pairs % nw == 0 and per_w % SC_CHUNK == 0 and steps % 2 == 0
    tok_chunk = SC_CHUNK // PAIRS_PER_PART
    ch_chunk = tok_chunk * WORD_LEN
    ngrp = SC_CHUNK // LANES
    n_idx = NPART * part_pairs
    mesh = plsc.VectorSubcoreMesh(core_axis_name="c", subcore_axis_name="s")

    @functools.partial(
        pl.kernel,
        mesh=mesh,
        compiler_params=pltpu.CompilerParams(use_tc_tiling_on_sc=False,
                                             needs_layout_passes=False),
        out_type=jax.ShapeDtypeStruct((n_idx, PAIR_DIM), jnp.float32),
        scratch_types=[
            pltpu.VMEM((ch_chunk,), jnp.int32),
            pltpu.VMEM((ch_chunk,), jnp.int32),
            pltpu.VMEM((SC_CHUNK,), jnp.int32),
            pltpu.VMEM((SC_CHUNK,), jnp.int32),
            pltpu.VMEM((SC_CHUNK, PAIR_DIM), jnp.float32),
            pltpu.VMEM((SC_CHUNK, PAIR_DIM), jnp.float32),
            pltpu.SemaphoreType.DMA,
            pltpu.SemaphoreType.DMA,
            pltpu.SemaphoreType.DMA,
            pltpu.SemaphoreType.DMA,
            pltpu.SemaphoreType.DMA,
        ],
    )
    def sc_gather(ch_hbm, table_hbm, out_hbm,
                  chv0, chv1, idxp0, idxp1, rows_v0, rows_v1,
                  sc0, sc1, so0, so1, sg):
        wid = lax.axis_index("s") * info.num_cores + lax.axis_index("c")
        w_ch = wid * steps * ch_chunk

        def ch_slice(c):
            return ch_hbm.at[pl.ds(w_ch + c * ch_chunk, ch_chunk)]

        def out_slice(base):
            return out_hbm.at[pl.ds(base, SC_CHUNK)]

        def compute_idx(g, c, chv, idxp):
            def grp_body(grp, carry):
                idxp[pl.ds(grp * LANES, LANES)] = _pair_indices(g, grp, chv)
                return carry
            lax.fori_loop(0, ngrp, grp_body, 0)

        pltpu.async_copy(rows_v0, out_slice(wid * per_w), so0)
        pltpu.async_copy(rows_v1, out_slice(wid * per_w + SC_CHUNK), so1)

        for g in range(NPART):
            w_out = g * part_pairs + wid * per_w

            pltpu.async_copy(ch_slice(0), chv0, sc0)
            pltpu.async_copy(ch_slice(1), chv1, sc1)

            def body(k, carry, g=g, w_out=w_out):
                c0 = 2 * k
                c1 = c0 + 1
                base0 = w_out + c0 * SC_CHUNK
                base1 = w_out + c1 * SC_CHUNK
                pre0 = jnp.minimum(c0 + 2, steps - 1)
                pre1 = jnp.minimum(c1 + 2, steps - 1)

                pltpu.make_async_copy(ch_slice(c0), chv0, sc0).wait()
                compute_idx(g, c0, chv0, idxp0)
                pltpu.async_copy(ch_slice(pre0), chv0, sc0)
                pltpu.make_async_copy(rows_v0, out_slice(base0), so0).wait()
                pltpu.async_copy(table_hbm.at[idxp0], rows_v0, sg).wait()
                pltpu.async_copy(rows_v0, out_slice(base0), so0)

                pltpu.make_async_copy(ch_slice(c1), chv1, sc1).wait()
                compute_idx(g, c1, chv1, idxp1)
                pltpu.async_copy(ch_slice(pre1), chv1, sc1)
                pltpu.make_async_copy(rows_v1, out_slice(base1), so1).wait()
                pltpu.async_copy(table_hbm.at[idxp1], rows_v1, sg).wait()
                pltpu.async_copy(rows_v1, out_slice(base1), so1)
                return carry

            lax.fori_loop(0, steps // 2, body, 0)

            pltpu.make_async_copy(ch_slice(0), chv0, sc0).wait()
            pltpu.make_async_copy(ch_slice(0), chv1, sc1).wait()

        pltpu.make_async_copy(rows_v0, out_slice(0), so0).wait()
        pltpu.make_async_copy(rows_v1, out_slice(0), so1).wait()

    return sc_gather


def _mlp_kernel(x_ref, w1_ref, b1_ref, w2_ref, b2_ref, out_ref):
    acc = jnp.dot(x_ref[0].astype(jnp.bfloat16), w1_ref[0],
                  preferred_element_type=jnp.float32)
    acc += jnp.dot(x_ref[1].astype(jnp.bfloat16), w1_ref[1],
                   preferred_element_type=jnp.float32)
    acc += jnp.dot(x_ref[2].astype(jnp.bfloat16), w1_ref[2],
                   preferred_element_type=jnp.float32)
    h = jax.nn.relu(acc + b1_ref[...])
    out = jnp.dot(h, w2_ref[...], preferred_element_type=jnp.float32)
    out_ref[...] = out + b2_ref[...]


def kernel(chars, emb, W1, b1, W2, b2):
    b, s, w = chars.shape
    n = b * s

    t2 = jnp.concatenate(
        [jnp.repeat(emb, CHAR_VOCAB, axis=0),
         jnp.tile(emb, (CHAR_VOCAB, 1))], axis=1)

    w1r = jnp.stack(
        [W1[0:128, :], W1[128:256, :],
         jnp.pad(W1[256:, :], ((0, 128 - (w * CHAR_EMB - 256)), (0, 0)))],
        axis=0).astype(jnp.bfloat16)
    b1r = b1.reshape(1, HIDDEN)
    b2r = b2.reshape(1, OUT_DIM)

    nh = n // 2
    hb = b // 2
    sc_gather = _make_sc_gather(nh)
    outs = []
    for r in range(2):
        ce = sc_gather(chars[r * hb:(r + 1) * hb].reshape(nh * w), t2)
        x3 = ce.reshape(NPART, nh, PART_CHARS * CHAR_EMB)
        grid = (nh // TOKEN_BLOCK,)
        outs.append(pl.pallas_call(
            _mlp_kernel,
            grid=grid,
            in_specs=[
                pl.BlockSpec((NPART, TOKEN_BLOCK, PART_CHARS * CHAR_EMB),
                             lambda i: (0, i, 0)),
                pl.BlockSpec((NPART, 128, HIDDEN), lambda i: (0, 0, 0)),
                pl.BlockSpec((1, HIDDEN), lambda i: (0, 0)),
                pl.BlockSpec((HIDDEN, OUT_DIM), lambda i: (0, 0)),
                pl.BlockSpec((1, OUT_DIM), lambda i: (0, 0)),
            ],
            out_specs=pl.BlockSpec((TOKEN_BLOCK, OUT_DIM), lambda i: (i, 0)),
            out_shape=jax.ShapeDtypeStruct((nh, OUT_DIM), jnp.float32),
        )(x3, w1r, b1r, W2, b2r))

    return jnp.concatenate(outs).reshape(b, s, OUT_DIM)


# speedup vs baseline: 1.1436x; 1.0909x over previous
import functools

import jax
import jax.numpy as jnp
from jax import lax
from jax.experimental import pallas as pl
from jax.experimental.pallas import tpu as pltpu
from jax.experimental.pallas import tpu_sc as plsc

CHAR_VOCAB = 256
CHAR_EMB = 16
WORD_LEN = 20
HIDDEN = 128
OUT_DIM = 64
NPART = 3
PART_CHARS = 8
PAIRS_PER_PART = PART_CHARS // 2
PAIR_DIM = 2 * CHAR_EMB

TOKEN_BLOCK = 4096
SC_CHUNK = 1600
LANES = 16


def _pair_indices(g, grp, chv_ref):
    ql = jax.lax.iota(jnp.int32, LANES) + grp * LANES
    lt = ql >> 2
    j = ql & 3
    if g == 0:
        col = 2 * j
    elif g == 1:
        col = PART_CHARS + 2 * j
    else:
        col = jnp.where(j < 2, 2 * PART_CHARS + 2 * j, 2 * j - 4)
    addr = lt * WORD_LEN + col
    hi = plsc.load_gather(chv_ref, [addr])
    lo = plsc.load_gather(chv_ref, [addr + 1])
    return (hi << 8) | lo


def _make_sc_gather(n_tok):
    info = plsc.get_sparse_core_info()
    nw = info.num_cores * info.num_subcores
    part_pairs = n_tok * PAIRS_PER_PART
    per_w = part_pairs // nw
    steps = per_w // SC_CHUNK
    assert part_pairs % nw == 0 and per_w % SC_CHUNK == 0 and steps % 2 == 0
    tok_chunk = SC_CHUNK // PAIRS_PER_PART
    ch_chunk = tok_chunk * WORD_LEN
    ngrp = SC_CHUNK // LANES
    n_idx = NPART * part_pairs
    mesh = plsc.VectorSubcoreMesh(core_axis_name="c", subcore_axis_name="s")

    @functools.partial(
        pl.kernel,
        mesh=mesh,
        compiler_params=pltpu.CompilerParams(use_tc_tiling_on_sc=False,
                                             needs_layout_passes=False),
        out_type=jax.ShapeDtypeStruct((n_idx, PAIR_DIM), jnp.float32),
        scratch_types=[
            pltpu.VMEM((ch_chunk,), jnp.int32),
            pltpu.VMEM((ch_chunk,), jnp.int32),
            pltpu.VMEM((SC_CHUNK,), jnp.int32),
            pltpu.VMEM((SC_CHUNK,), jnp.int32),
            pltpu.VMEM((SC_CHUNK, PAIR_DIM), jnp.float32),
            pltpu.VMEM((SC_CHUNK, PAIR_DIM), jnp.float32),
            pltpu.SemaphoreType.DMA,
            pltpu.SemaphoreType.DMA,
            pltpu.SemaphoreType.DMA,
            pltpu.SemaphoreType.DMA,
            pltpu.SemaphoreType.DMA,
        ],
    )
    def sc_gather(ch_hbm, table_hbm, out_hbm,
                  chv0, chv1, idxp0, idxp1, rows_v0, rows_v1,
                  sc0, sc1, so0, so1, sg):
        wid = lax.axis_index("s") * info.num_cores + lax.axis_index("c")
        w_ch = wid * steps * ch_chunk

        def ch_slice(c):
            return ch_hbm.at[pl.ds(w_ch + c * ch_chunk, ch_chunk)]

        def out_slice(base):
            return out_hbm.at[pl.ds(base, SC_CHUNK)]

        def compute_idx(g, c, chv, idxp):
            def grp_body(grp, carry):
                idxp[pl.ds(grp * LANES, LANES)] = _pair_indices(g, grp, chv)
                return carry
            lax.fori_loop(0, ngrp, grp_body, 0)

        pltpu.async_copy(rows_v0, out_slice(wid * per_w), so0)
        pltpu.async_copy(rows_v1, out_slice(wid * per_w + SC_CHUNK), so1)

        for g in range(NPART):
            w_out = g * part_pairs + wid * per_w

            pltpu.async_copy(ch_slice(0), chv0, sc0)
            pltpu.async_copy(ch_slice(1), chv1, sc1)

            def body(k, carry, g=g, w_out=w_out):
                c0 = 2 * k
                c1 = c0 + 1
                base0 = w_out + c0 * SC_CHUNK
                base1 = w_out + c1 * SC_CHUNK
                pre0 = jnp.minimum(c0 + 2, steps - 1)
                pre1 = jnp.minimum(c1 + 2, steps - 1)

                pltpu.make_async_copy(ch_slice(c0), chv0, sc0).wait()
                compute_idx(g, c0, chv0, idxp0)
                pltpu.async_copy(ch_slice(pre0), chv0, sc0)
                pltpu.make_async_copy(rows_v0, out_slice(base0), so0).wait()
                pltpu.async_copy(table_hbm.at[idxp0], rows_v0, sg).wait()
                pltpu.async_copy(rows_v0, out_slice(base0), so0)

                pltpu.make_async_copy(ch_slice(c1), chv1, sc1).wait()
                compute_idx(g, c1, chv1, idxp1)
                pltpu.async_copy(ch_slice(pre1), chv1, sc1)
                pltpu.make_async_copy(rows_v1, out_slice(base1), so1).wait()
                pltpu.async_copy(table_hbm.at[idxp1], rows_v1, sg).wait()
                pltpu.async_copy(rows_v1, out_slice(base1), so1)
                return carry

            lax.fori_loop(0, steps // 2, body, 0)

            pltpu.make_async_copy(ch_slice(0), chv0, sc0).wait()
            pltpu.make_async_copy(ch_slice(0), chv1, sc1).wait()

        pltpu.make_async_copy(rows_v0, out_slice(0), so0).wait()
        pltpu.make_async_copy(rows_v1, out_slice(0), so1).wait()

    return sc_gather


def _mlp_kernel(x_ref, w1_ref, b1_ref, w2_ref, b2_ref, out_ref):
    acc = jnp.dot(x_ref[0].astype(jnp.bfloat16), w1_ref[0],
                  preferred_element_type=jnp.float32)
    acc += jnp.dot(x_ref[1].astype(jnp.bfloat16), w1_ref[1],
                   preferred_element_type=jnp.float32)
    acc += jnp.dot(x_ref[2].astype(jnp.bfloat16), w1_ref[2],
                   preferred_element_type=jnp.float32)
    h = jax.nn.relu(acc + b1_ref[...])
    out = jnp.dot(h, w2_ref[...], preferred_element_type=jnp.float32)
    out_ref[...] = out + b2_ref[...]


def kernel(chars, emb, W1, b1, W2, b2):
    b, s, w = chars.shape
    n = b * s

    t2 = jnp.concatenate(
        [jnp.repeat(emb, CHAR_VOCAB, axis=0),
         jnp.tile(emb, (CHAR_VOCAB, 1))], axis=1)

    ce = _make_sc_gather(n)(chars.reshape(n * w), t2)
    x3 = ce.reshape(NPART, n, PART_CHARS * CHAR_EMB)

    w1r = jnp.stack(
        [W1[0:128, :], W1[128:256, :],
         jnp.pad(W1[256:, :], ((0, 128 - (w * CHAR_EMB - 256)), (0, 0)))],
        axis=0).astype(jnp.bfloat16)

    grid = (n // TOKEN_BLOCK,)
    out = pl.pallas_call(
        _mlp_kernel,
        grid=grid,
        in_specs=[
            pl.BlockSpec((NPART, TOKEN_BLOCK, PART_CHARS * CHAR_EMB),
                         lambda i: (0, i, 0)),
            pl.BlockSpec((NPART, 128, HIDDEN), lambda i: (0, 0, 0)),
            pl.BlockSpec((1, HIDDEN), lambda i: (0, 0)),
            pl.BlockSpec((HIDDEN, OUT_DIM), lambda i: (0, 0)),
            pl.BlockSpec((1, OUT_DIM), lambda i: (0, 0)),
        ],
        out_specs=pl.BlockSpec((TOKEN_BLOCK, OUT_DIM), lambda i: (i, 0)),
        out_shape=jax.ShapeDtypeStruct((n, OUT_DIM), jnp.float32),
    )(x3, w1r, b1.reshape(1, HIDDEN), W2, b2.reshape(1, OUT_DIM))

    return out.reshape(b, s, OUT_DIM)


# speedup vs baseline: 1.1535x; 1.0087x over previous
import functools

import jax
import jax.numpy as jnp
from jax import lax
from jax.experimental import pallas as pl
from jax.experimental.pallas import tpu as pltpu
from jax.experimental.pallas import tpu_sc as plsc

CHAR_VOCAB = 256
CHAR_EMB = 16
WORD_LEN = 20
HIDDEN = 128
OUT_DIM = 64
NPART = 3
PART_CHARS = 8
PAIRS_PER_PART = PART_CHARS // 2
PAIR_DIM = 2 * CHAR_EMB

TOKEN_BLOCK = 8192
SC_CHUNK = 1600
LANES = 16


def _pair_indices(g, grp, chv_ref):
    ql = jax.lax.iota(jnp.int32, LANES) + grp * LANES
    lt = ql >> 2
    j = ql & 3
    if g == 0:
        col = 2 * j
    elif g == 1:
        col = PART_CHARS + 2 * j
    else:
        col = jnp.where(j < 2, 2 * PART_CHARS + 2 * j, 2 * j - 4)
    addr = lt * WORD_LEN + col
    hi = plsc.load_gather(chv_ref, [addr])
    lo = plsc.load_gather(chv_ref, [addr + 1])
    return (hi << 8) | lo


def _make_sc_gather(n_tok):
    info = plsc.get_sparse_core_info()
    nw = info.num_cores * info.num_subcores
    part_pairs = n_tok * PAIRS_PER_PART
    per_w = part_pairs // nw
    steps = per_w // SC_CHUNK
    assert part_pairs % nw == 0 and per_w % SC_CHUNK == 0 and steps % 2 == 0
    tok_chunk = SC_CHUNK // PAIRS_PER_PART
    ch_chunk = tok_chunk * WORD_LEN
    ngrp = SC_CHUNK // LANES
    n_idx = NPART * part_pairs
    mesh = plsc.VectorSubcoreMesh(core_axis_name="c", subcore_axis_name="s")

    @functools.partial(
        pl.kernel,
        mesh=mesh,
        compiler_params=pltpu.CompilerParams(use_tc_tiling_on_sc=False,
                                             needs_layout_passes=False),
        out_type=jax.ShapeDtypeStruct((n_idx, PAIR_DIM), jnp.float32),
        scratch_types=[
            pltpu.VMEM((ch_chunk,), jnp.int32),
            pltpu.VMEM((ch_chunk,), jnp.int32),
            pltpu.VMEM((SC_CHUNK,), jnp.int32),
            pltpu.VMEM((SC_CHUNK,), jnp.int32),
            pltpu.VMEM((SC_CHUNK, PAIR_DIM), jnp.float32),
            pltpu.VMEM((SC_CHUNK, PAIR_DIM), jnp.float32),
            pltpu.SemaphoreType.DMA,
            pltpu.SemaphoreType.DMA,
            pltpu.SemaphoreType.DMA,
            pltpu.SemaphoreType.DMA,
            pltpu.SemaphoreType.DMA,
        ],
    )
    def sc_gather(ch_hbm, table_hbm, out_hbm,
                  chv0, chv1, idxp0, idxp1, rows_v0, rows_v1,
                  sc0, sc1, so0, so1, sg):
        wid = lax.axis_index("s") * info.num_cores + lax.axis_index("c")
        w_ch = wid * steps * ch_chunk

        def ch_slice(c):
            return ch_hbm.at[pl.ds(w_ch + c * ch_chunk, ch_chunk)]

        def out_slice(base):
            return out_hbm.at[pl.ds(base, SC_CHUNK)]

        def compute_idx(g, c, chv, idxp):
            def grp_body(grp, carry):
                idxp[pl.ds(grp * LANES, LANES)] = _pair_indices(g, grp, chv)
                return carry
            lax.fori_loop(0, ngrp, grp_body, 0)

        pltpu.async_copy(rows_v0, out_slice(wid * per_w), so0)
        pltpu.async_copy(rows_v1, out_slice(wid * per_w + SC_CHUNK), so1)

        for g in range(NPART):
            w_out = g * part_pairs + wid * per_w

            pltpu.async_copy(ch_slice(0), chv0, sc0)
            pltpu.async_copy(ch_slice(1), chv1, sc1)

            def body(k, carry, g=g, w_out=w_out):
                c0 = 2 * k
                c1 = c0 + 1
                base0 = w_out + c0 * SC_CHUNK
                base1 = w_out + c1 * SC_CHUNK
                pre0 = jnp.minimum(c0 + 2, steps - 1)
                pre1 = jnp.minimum(c1 + 2, steps - 1)

                pltpu.make_async_copy(ch_slice(c0), chv0, sc0).wait()
                compute_idx(g, c0, chv0, idxp0)
                pltpu.async_copy(ch_slice(pre0), chv0, sc0)
                pltpu.make_async_copy(rows_v0, out_slice(base0), so0).wait()
                pltpu.async_copy(table_hbm.at[idxp0], rows_v0, sg).wait()
                pltpu.async_copy(rows_v0, out_slice(base0), so0)

                pltpu.make_async_copy(ch_slice(c1), chv1, sc1).wait()
                compute_idx(g, c1, chv1, idxp1)
                pltpu.async_copy(ch_slice(pre1), chv1, sc1)
                pltpu.make_async_copy(rows_v1, out_slice(base1), so1).wait()
                pltpu.async_copy(table_hbm.at[idxp1], rows_v1, sg).wait()
                pltpu.async_copy(rows_v1, out_slice(base1), so1)
                return carry

            lax.fori_loop(0, steps // 2, body, 0)

            pltpu.make_async_copy(ch_slice(0), chv0, sc0).wait()
            pltpu.make_async_copy(ch_slice(0), chv1, sc1).wait()

        pltpu.make_async_copy(rows_v0, out_slice(0), so0).wait()
        pltpu.make_async_copy(rows_v1, out_slice(0), so1).wait()

    return sc_gather


def _mlp_kernel(x_ref, w1_ref, b1_ref, w2_ref, b2_ref, out_ref):
    acc = jnp.dot(x_ref[0].astype(jnp.bfloat16), w1_ref[0],
                  preferred_element_type=jnp.float32)
    acc += jnp.dot(x_ref[1].astype(jnp.bfloat16), w1_ref[1],
                   preferred_element_type=jnp.float32)
    acc += jnp.dot(x_ref[2].astype(jnp.bfloat16), w1_ref[2],
                   preferred_element_type=jnp.float32)
    h = jax.nn.relu(acc + b1_ref[...])
    out = jnp.dot(h, w2_ref[...], preferred_element_type=jnp.float32)
    out_ref[...] = out + b2_ref[...]


def kernel(chars, emb, W1, b1, W2, b2):
    b, s, w = chars.shape
    n = b * s

    t2 = jnp.concatenate(
        [jnp.repeat(emb, CHAR_VOCAB, axis=0),
         jnp.tile(emb, (CHAR_VOCAB, 1))], axis=1)

    ce = _make_sc_gather(n)(chars.reshape(n * w), t2)
    x3 = ce.reshape(NPART, n, PART_CHARS * CHAR_EMB)

    w1r = jnp.stack(
        [W1[0:128, :], W1[128:256, :],
         jnp.pad(W1[256:, :], ((0, 128 - (w * CHAR_EMB - 256)), (0, 0)))],
        axis=0).astype(jnp.bfloat16)

    grid = (n // TOKEN_BLOCK,)
    out = pl.pallas_call(
        _mlp_kernel,
        grid=grid,
        in_specs=[
            pl.BlockSpec((NPART, TOKEN_BLOCK, PART_CHARS * CHAR_EMB),
                         lambda i: (0, i, 0)),
            pl.BlockSpec((NPART, 128, HIDDEN), lambda i: (0, 0, 0)),
            pl.BlockSpec((1, HIDDEN), lambda i: (0, 0)),
            pl.BlockSpec((HIDDEN, OUT_DIM), lambda i: (0, 0)),
            pl.BlockSpec((1, OUT_DIM), lambda i: (0, 0)),
        ],
        out_specs=pl.BlockSpec((TOKEN_BLOCK, OUT_DIM), lambda i: (i, 0)),
        out_shape=jax.ShapeDtypeStruct((n, OUT_DIM), jnp.float32),
    )(x3, w1r, b1.reshape(1, HIDDEN), W2, b2.reshape(1, OUT_DIM))

    return out.reshape(b, s, OUT_DIM)
